# baseline (speedup 1.0000x reference)
import functools

import jax
import jax.numpy as jnp
import numpy as np
from jax.experimental import pallas as pl
from jax.experimental.pallas import tpu as pltpu

_K = 20
_QB = 16
_R = 6
_PAD_BIG = 1e30
_IMAX = 2**31 - 1


def _cost_kernel(x_ref, p_ref, out_ref, f_ref, k_ref):
    i = pl.program_id(0)

    @pl.when(i == 0)
    def _build_features():
        px = p_ref[0:1, :]
        py = p_ref[1:2, :]
        pz = p_ref[2:3, :]
        pad = p_ref[3:4, :]
        pxb = px.astype(jnp.bfloat16).astype(jnp.float32)
        pyb = py.astype(jnp.bfloat16).astype(jnp.float32)
        pzb = pz.astype(jnp.bfloat16).astype(jnp.float32)
        f_ref[0:1, :] = (px * px + py * py) + pz * pz + pad
        f_ref[1:2, :] = pxb
        f_ref[2:3, :] = pyb
        f_ref[3:4, :] = pzb
        f_ref[4:5, :] = pxb * pxb
        f_ref[5:6, :] = pxb * pyb
        f_ref[6:7, :] = pyb * pyb
        f_ref[7:8, :] = pxb * pzb
        f_ref[8:9, :] = pyb * pzb

    x = x_ref[0]
    x0 = x[:, 0:1]
    x1 = x[:, 1:2]
    x2 = x[:, 2:3]
    q2 = (x0 * x0 + x1 * x1) + x2 * x2

    xb = x.astype(jnp.bfloat16)
    rhs = jnp.concatenate(
        [p_ref[0:1, :], p_ref[1:2, :], p_ref[2:3, :]], axis=0
    ).astype(jnp.bfloat16)
    mm = jax.lax.dot_general(xb, rhs, (((1,), (0,)), ((), ())),
                             preferred_element_type=jnp.float32)
    dist2 = (q2 - 2.0 * mm) + f_ref[0:1, :]
    bits = jax.lax.bitcast_convert_type(dist2, jnp.int32)
    k_ref[...] = jnp.where(bits < 0, bits ^ 0x7FFFFFFF, bits)

    nw = k_ref.shape[1] // 128

    def ins(g, rs):
        off = pl.multiple_of(g * 128, 128)
        v = k_ref[:, pl.ds(off, 128)]
        out = []
        for r in rs:
            out.append(jnp.minimum(r, v))
            v = jnp.maximum(r, v)
        return tuple(out)

    rs0 = tuple(jnp.full((_QB, 128), _IMAX, jnp.int32) for _ in range(_R))
    rs = jax.lax.fori_loop(0, nw, ins, rs0)
    cand = jnp.concatenate(rs, axis=1)

    clo = jnp.min(cand, axis=1, keepdims=True) - 1
    chi = jnp.max(cand, axis=1, keepdims=True)

    def cbis(_, st):
        lo, hi = st
        mid = (lo >> 1) + (hi >> 1) + (lo & hi & 1)
        c = jnp.sum((cand <= mid).astype(jnp.int32), axis=1, keepdims=True)
        ge = c >= _K
        return jnp.where(ge, lo, mid), jnp.where(ge, mid, hi)

    clo, chi = jax.lax.fori_loop(0, 32, cbis, (clo, chi))
    t_hat = chi

    def vcnt(g, acc):
        off = pl.multiple_of(g * 128, 128)
        v = k_ref[:, pl.ds(off, 128)]
        return acc + (v <= t_hat).astype(jnp.int32)

    lanecnt = jax.lax.fori_loop(0, nw, vcnt,
                                jnp.zeros((_QB, 128), jnp.int32))
    ok = jnp.all(lanecnt <= _R)

    def full_bisect():
        lo = clo * 0 + (jnp.min(cand, axis=1, keepdims=True) - 1)
        hi = t_hat

        def bis(_, st):
            lo, hi = st
            mid = (lo >> 1) + (hi >> 1) + (lo & hi & 1)
            c = jnp.sum((k_ref[...] <= mid).astype(jnp.int32), axis=1,
                        keepdims=True)
            ge = c >= _K
            return jnp.where(ge, lo, mid), jnp.where(ge, mid, hi)

        lo, hi = jax.lax.fori_loop(0, 32, bis, (lo, hi))
        return hi

    tkey = jax.lax.cond(ok, lambda: t_hat, full_bisect)

    keys = k_ref[...]
    mask_lt = keys < tkey
    cnt_lt = jnp.sum(mask_lt.astype(jnp.int32), axis=1, keepdims=True)
    need0 = _K - cnt_lt
    iota = jax.lax.broadcasted_iota(jnp.int32, keys.shape, 1)
    eq = keys == tkey

    ieq = jnp.where(eq, iota, _IMAX)

    def corr(j, ti):
        cand = jnp.where(ieq > ti, ieq, _IMAX)
        imin = jnp.min(cand, axis=1, keepdims=True)
        return jnp.where(need0 > j, imin, ti)

    ti2 = jax.lax.fori_loop(0, 2, corr, jnp.full_like(cnt_lt, -1))
    ti = jax.lax.cond(jnp.any(need0 > 2),
                      lambda: jax.lax.fori_loop(2, 20, corr, ti2),
                      lambda: ti2)
    maskf = (mask_lt | (eq & (iota <= ti))).astype(jnp.float32)

    def msum(row):
        return jnp.sum(maskf * row, axis=1, keepdims=True)

    cnt = jnp.full((_QB, 1), float(_K), jnp.float32)
    sx = msum(f_ref[1:2, :])
    sy = msum(f_ref[2:3, :])
    sz = msum(f_ref[3:4, :])
    sxx = msum(f_ref[4:5, :])
    sxy = msum(f_ref[5:6, :])
    syy = msum(f_ref[6:7, :])
    sxz = msum(f_ref[7:8, :])
    syz = msum(f_ref[8:9, :])

    a00, a01, a02 = sxx, sxy, sx
    a10, a11, a12 = sxy, syy, sy
    a20, a21, a22 = sx, sy, cnt
    b0, b1, b2 = sxz, syz, sz

    m0, m1, m2 = jnp.abs(a00), jnp.abs(a10), jnp.abs(a20)
    p1 = (m1 > m0) & (m1 >= m2)

    def sw(c, u, v):
        return jnp.where(c, v, u), jnp.where(c, u, v)

    a00, a10 = sw(p1, a00, a10)
    a01, a11 = sw(p1, a01, a11)
    a02, a12 = sw(p1, a02, a12)
    b0, b1 = sw(p1, b0, b1)
    p2 = (m2 > m0) & (m2 > m1)
    a00, a20 = sw(p2, a00, a20)
    a01, a21 = sw(p2, a01, a21)
    a02, a22 = sw(p2, a02, a22)
    b0, b2 = sw(p2, b0, b2)

    l10 = a10 / a00
    l20 = a20 / a00
    a11 = a11 + (-(l10 * a01))
    a12 = a12 + (-(l10 * a02))
    a21 = a21 + (-(l20 * a01))
    a22 = a22 + (-(l20 * a02))
    psw = jnp.abs(a21) > jnp.abs(a11)
    a11, a21 = sw(psw, a11, a21)
    a12, a22 = sw(psw, a12, a22)
    b1, b2 = sw(psw, b1, b2)
    l10, l20 = sw(psw, l10, l20)
    l21 = a21 / a11
    a22 = a22 + (-(l21 * a12))

    y0 = b0
    y1 = b1 - l10 * y0
    y2 = (b2 - l20 * y0) - l21 * y1
    w2 = y2 / a22
    w1 = (y1 - a12 * w2) / a11
    w0 = ((y0 - a01 * w1) - a02 * w2) / a00

    pn_d = x0 * w0 + x1 * w1 - x2 + w2
    nn = w0 * w0 + w1 * w1 + 1.0
    r = pn_d / nn
    closeness = r * pn_d
    height = jnp.exp(x2 + r)

    def sig(v):
        return 1.0 / (1.0 + jnp.exp(-v))

    boundary = (sig((x0 - 5.0) * 10.0) + 1.0 - sig((x0 + 5.0) * 10.0)
                + sig((x1 - 5.0) * 10.0) + 1.0 - sig((x1 + 5.0) * 10.0))
    out_ref[0] = closeness + height + boundary


@jax.jit
def _run(x, p):
    nq = x.shape[0]
    npad = p.shape[1]
    grid = nq // _QB
    out = pl.pallas_call(
        _cost_kernel,
        grid=(grid,),
        in_specs=[
            pl.BlockSpec((1, _QB, 3), lambda i: (i, 0, 0)),
            pl.BlockSpec((8, npad), lambda i: (0, 0)),
        ],
        out_specs=pl.BlockSpec((1, _QB, 1), lambda i: (i, 0, 0)),
        out_shape=jax.ShapeDtypeStruct((grid, _QB, 1), jnp.float32),
        scratch_shapes=[
            pltpu.VMEM((16, npad), jnp.float32),
            pltpu.VMEM((_QB, npad), jnp.int32),
        ],
    )(x.reshape(grid, _QB, 3), p)
    return out.reshape(nq)


def kernel(xt, dataset):
    shape = xt.shape[:-1]
    n = int(np.prod(shape))
    x = xt.reshape(n, 3).astype(jnp.float32)

    nq = ((n + _QB - 1) // _QB) * _QB
    if nq != n:
        x = jnp.pad(x, ((0, nq - n), (0, 0)))

    d = dataset.shape[0]
    npad = ((d + 127) // 128) * 128
    p = jnp.zeros((8, npad), jnp.float32)
    p = p.at[0:3, :d].set(dataset.T.astype(jnp.float32))
    p = p.at[3, d:].set(_PAD_BIG)

    out = _run(x, p)
    return out[:n].reshape(shape)
